# baseline (speedup 1.0000x reference)
import functools

import jax
import jax.numpy as jnp
from jax import lax
from jax.experimental import pallas as pl
from jax.experimental.pallas import tpu as pltpu
from jax.experimental.pallas import tpu_sc as plsc

_MAXLEN = 200
_EMBED = 64
_BATCH = 1024
_VOCAB = 1000000
_NC, _NS, _L = 2, 16, 16
_NW = _NC * _NS
_TOK_PER_W = _BATCH * _MAXLEN // _NW
_STEP = 128
_NSTEPS = _TOK_PER_W // _STEP
_SPAN = 8192
_RROWS = 122 * 4096 + 576


def _repack_body(tt_ref, out_ref):
    out_ref[:, 0:_EMBED] = tt_ref[:, 0:_SPAN // 2].T
    out_ref[:, _EMBED:128] = tt_ref[:, _SPAN // 2:_SPAN].T


_repack_kernel = pl.pallas_call(
    _repack_body,
    out_shape=jax.ShapeDtypeStruct((_RROWS, 128), jnp.float32),
    grid=(pl.cdiv(_VOCAB, _SPAN),),
    in_specs=[pl.BlockSpec((_EMBED, _SPAN), lambda i: (0, i))],
    out_specs=pl.BlockSpec((_SPAN // 2, 128), lambda i: (i, 0)),
)


@functools.partial(
    pl.kernel,
    out_type=jax.ShapeDtypeStruct((_BATCH * _MAXLEN, _EMBED), jnp.float32),
    mesh=plsc.VectorSubcoreMesh(core_axis_name="c", subcore_axis_name="s"),
    scratch_types=[
        pltpu.VMEM((_NSTEPS, _STEP), jnp.int32),
        pltpu.VMEM((_NSTEPS, _STEP), jnp.int32),
        pltpu.VMEM((_MAXLEN, _EMBED), jnp.float32),
        pltpu.VMEM((_STEP, 128), jnp.float32),
        pltpu.VMEM((_STEP, 128), jnp.float32),
        pltpu.VMEM((_STEP, 128), jnp.float32),
        pltpu.VMEM((_STEP, _EMBED), jnp.float32),
        pltpu.VMEM((_STEP, _EMBED), jnp.float32),
        pltpu.SemaphoreType.DMA,
        pltpu.SemaphoreType.DMA,
        pltpu.SemaphoreType.DMA,
        pltpu.SemaphoreType.DMA,
        pltpu.SemaphoreType.DMA,
    ],
    compiler_params=pltpu.CompilerParams(use_tc_tiling_on_sc=True),
)
def _emb_kernel(x_hbm, table_hbm, pos_hbm, out_hbm,
                idx_v, idx_g, pos_v, buf0, buf1, buf2, st0, st1,
                gsem0, gsem1, gsem2, ssem0, ssem1):
    wid = lax.axis_index("s") * _NC + lax.axis_index("c")
    pltpu.sync_copy(x_hbm.at[wid], idx_v)
    pltpu.sync_copy(pos_hbm, pos_v)

    @plsc.parallel_loop(0, _NSTEPS, step=1, unroll=2)
    def _(j):
        for k in range(_STEP // _L):
            sl = pl.ds(k * _L, _L)
            v = idx_v[j, sl]
            idx_g[j, sl] = jnp.left_shift(jnp.right_shift(v, 13), 12) + (v & 4095)

    bufs = (buf0, buf1, buf2)
    sts = (st0, st1)
    gsems = (gsem0, gsem1, gsem2)
    ssems = (ssem0, ssem1)
    out_base = wid * _TOK_PER_W

    def issue_gather(g, buf, gsem):
        pltpu.async_copy(table_hbm.at[idx_g.at[g]], buf, gsem)

    def wait_gather(g, buf, gsem):
        pltpu.make_async_copy(table_hbm.at[idx_g.at[g]], buf, gsem).wait()

    def wait_store(st, ssem):
        pltpu.make_async_copy(st, out_hbm.at[pl.ds(0, _STEP)], ssem).wait()

    def add_and_store(g, cur, st, ssem):
        base_t = lax.rem(g * _STEP, _MAXLEN)

        @plsc.parallel_loop(0, _STEP, step=1, unroll=8)
        def _(r):
            t = base_t + r
            t = jnp.where(t >= _MAXLEN, t - _MAXLEN, t)
            vr = idx_v[g, pl.ds(r, 1)]
            half = (jnp.right_shift(vr[0], 12) & 1) * _EMBED
            for k in range(_EMBED // _L):
                st[r, pl.ds(k * _L, _L)] = (
                    cur[r, pl.ds(half + k * _L, _L)]
                    + pos_v[t, pl.ds(k * _L, _L)]
                )

        pltpu.async_copy(
            st, out_hbm.at[pl.ds(out_base + g * _STEP, _STEP)], ssem
        )

    for g0 in range(2):
        issue_gather(g0, bufs[g0], gsems[g0])

    def outer(i, carry):
        for b in range(6):
            g = 6 * i + b
            cur = bufs[b % 3]
            st = sts[b % 2]

            @pl.when(g + 2 < _NSTEPS)
            def _():
                issue_gather(g + 2, bufs[(b + 2) % 3], gsems[(b + 2) % 3])

            @pl.when(g >= 2)
            def _():
                wait_store(st, ssems[b % 2])

            wait_gather(g, cur, gsems[b % 3])
            add_and_store(g, cur, st, ssems[b % 2])
        return carry

    lax.fori_loop(0, (_NSTEPS - 2) // 6, outer, 0)
    for g in (_NSTEPS - 2, _NSTEPS - 1):
        b = g % 6
        wait_store(sts[b % 2], ssems[b % 2])
        wait_gather(g, bufs[b % 3], gsems[b % 3])
        add_and_store(g, bufs[b % 3], sts[b % 2], ssems[b % 2])
    wait_store(st0, ssem0)
    wait_store(st1, ssem1)


def kernel(x, token_table, pos_table):
    x_r = x.reshape(_NW, _NSTEPS, _STEP)
    repacked = _repack_kernel(token_table.T)
    out = _emb_kernel(x_r, repacked, pos_table)
    return out.reshape(_BATCH, _MAXLEN, _EMBED)


# speedup vs baseline: 1.0239x; 1.0239x over previous
import functools

import jax
import jax.numpy as jnp
from jax import lax
from jax.experimental import pallas as pl
from jax.experimental.pallas import tpu as pltpu
from jax.experimental.pallas import tpu_sc as plsc

_MAXLEN = 200
_EMBED = 64
_BATCH = 1024
_VOCAB = 1000000
_NC, _NS, _L = 2, 16, 16
_NW = _NC * _NS
_TOK_PER_W = _BATCH * _MAXLEN // _NW
_STEP = 128
_NSTEPS = _TOK_PER_W // _STEP
_SPAN = 8192
_RROWS = 122 * 4096 + 576


def _repack_body(tt_ref, out_ref):
    out_ref[:, 0:_EMBED] = tt_ref[:, 0:_SPAN // 2].T
    out_ref[:, _EMBED:128] = tt_ref[:, _SPAN // 2:_SPAN].T


_repack_kernel = pl.pallas_call(
    _repack_body,
    out_shape=jax.ShapeDtypeStruct((_RROWS, 128), jnp.float32),
    grid=(pl.cdiv(_VOCAB, _SPAN),),
    in_specs=[pl.BlockSpec((_EMBED, _SPAN), lambda i: (0, i))],
    out_specs=pl.BlockSpec((_SPAN // 2, 128), lambda i: (i, 0)),
)


@functools.partial(
    pl.kernel,
    out_type=jax.ShapeDtypeStruct((_BATCH * _MAXLEN, _EMBED), jnp.float32),
    mesh=plsc.VectorSubcoreMesh(core_axis_name="c", subcore_axis_name="s"),
    scratch_types=[
        pltpu.VMEM((_NSTEPS, _STEP), jnp.int32),
        pltpu.VMEM((_NSTEPS, _STEP), jnp.int32),
        pltpu.VMEM((_MAXLEN, _EMBED), jnp.float32),
        pltpu.VMEM((_STEP, 128), jnp.float32),
        pltpu.VMEM((_STEP, 128), jnp.float32),
        pltpu.VMEM((_STEP, _EMBED), jnp.float32),
        pltpu.VMEM((_STEP, _EMBED), jnp.float32),
        pltpu.SemaphoreType.DMA,
        pltpu.SemaphoreType.DMA,
        pltpu.SemaphoreType.DMA,
        pltpu.SemaphoreType.DMA,
    ],
    compiler_params=pltpu.CompilerParams(use_tc_tiling_on_sc=True),
)
def _emb_kernel(x_hbm, table_hbm, pos_hbm, out_hbm,
                idx_v, idx_g, pos_v, buf0, buf1, st0, st1,
                gsem0, gsem1, ssem0, ssem1):
    wid = lax.axis_index("s") * _NC + lax.axis_index("c")
    pltpu.sync_copy(x_hbm.at[wid], idx_v)
    pltpu.sync_copy(pos_hbm, pos_v)

    @plsc.parallel_loop(0, _NSTEPS, step=1, unroll=2)
    def _(j):
        for k in range(_STEP // _L):
            sl = pl.ds(k * _L, _L)
            v = idx_v[j, sl]
            idx_g[j, sl] = jnp.left_shift(jnp.right_shift(v, 13), 12) + (v & 4095)

    bufs = (buf0, buf1)
    sts = (st0, st1)
    gsems = (gsem0, gsem1)
    ssems = (ssem0, ssem1)
    out_base = wid * _TOK_PER_W

    def issue_gather(g, buf, gsem):
        pltpu.async_copy(table_hbm.at[idx_g.at[g]], buf, gsem)

    def wait_gather(g, buf, gsem):
        pltpu.make_async_copy(table_hbm.at[idx_g.at[g]], buf, gsem).wait()

    def wait_store(st, ssem):
        pltpu.make_async_copy(st, out_hbm.at[pl.ds(0, _STEP)], ssem).wait()

    def add_and_store(g, cur, st, ssem):
        base_t = lax.rem(g * _STEP, _MAXLEN)

        @plsc.parallel_loop(0, _STEP, step=1, unroll=8)
        def _(r):
            t = base_t + r
            t = jnp.where(t >= _MAXLEN, t - _MAXLEN, t)
            vr = idx_v[g, pl.ds(r, 1)]
            half = (jnp.right_shift(vr[0], 12) & 1) * _EMBED
            for k in range(_EMBED // _L):
                st[r, pl.ds(k * _L, _L)] = (
                    cur[r, pl.ds(half + k * _L, _L)]
                    + pos_v[t, pl.ds(k * _L, _L)]
                )

        pltpu.async_copy(
            st, out_hbm.at[pl.ds(out_base + g * _STEP, _STEP)], ssem
        )

    issue_gather(0, buf0, gsem0)

    def outer(i, carry):
        for b in range(2):
            g = 2 * i + b
            cur, nxt = bufs[b], bufs[1 - b]
            st = sts[b]

            @pl.when(g + 1 < _NSTEPS)
            def _():
                issue_gather(g + 1, nxt, gsems[1 - b])

            @pl.when(g >= 2)
            def _():
                wait_store(st, ssems[b])

            wait_gather(g, cur, gsems[b])
            add_and_store(g, cur, st, ssems[b])
        return carry

    lax.fori_loop(0, _NSTEPS // 2, outer, 0)
    wait_store(st0, ssem0)
    wait_store(st1, ssem1)


def kernel(x, token_table, pos_table):
    x_r = x.reshape(_NW, _NSTEPS, _STEP)
    repacked = _repack_kernel(token_table.T)
    out = _emb_kernel(x_r, repacked, pos_table)
    return out.reshape(_BATCH, _MAXLEN, _EMBED)


# speedup vs baseline: 1.1093x; 1.0834x over previous
import functools

import jax
import jax.numpy as jnp
from jax import lax
from jax.experimental import pallas as pl
from jax.experimental.pallas import tpu as pltpu
from jax.experimental.pallas import tpu_sc as plsc

_MAXLEN = 200
_EMBED = 64
_BATCH = 1024
_VOCAB = 1000000
_NC, _NS, _L = 2, 16, 16
_NW = _NC * _NS
_TOK_PER_W = _BATCH * _MAXLEN // _NW
_STEP = 128
_NSTEPS = _TOK_PER_W // _STEP
_SPAN = 16384
_RROWS = 61 * 8192 + 576


def _repack_body(tt_ref, out_ref):
    out_ref[:, 0:_EMBED] = tt_ref[:, 0:_SPAN // 2].T
    out_ref[:, _EMBED:128] = tt_ref[:, _SPAN // 2:_SPAN].T


_repack_kernel = pl.pallas_call(
    _repack_body,
    out_shape=jax.ShapeDtypeStruct((_RROWS, 128), jnp.float32),
    grid=(pl.cdiv(_VOCAB, _SPAN),),
    in_specs=[pl.BlockSpec((_EMBED, _SPAN), lambda i: (0, i))],
    out_specs=pl.BlockSpec((_SPAN // 2, 128), lambda i: (i, 0)),
)


@functools.partial(
    pl.kernel,
    out_type=jax.ShapeDtypeStruct((_BATCH * _MAXLEN, _EMBED), jnp.float32),
    mesh=plsc.VectorSubcoreMesh(core_axis_name="c", subcore_axis_name="s"),
    scratch_types=[
        pltpu.VMEM((_NSTEPS, _STEP), jnp.int32),
        pltpu.VMEM((_NSTEPS, _STEP), jnp.int32),
        pltpu.VMEM((_MAXLEN, _EMBED), jnp.float32),
        pltpu.VMEM((_STEP, 128), jnp.float32),
        pltpu.VMEM((_STEP, 128), jnp.float32),
        pltpu.VMEM((_STEP, _EMBED), jnp.float32),
        pltpu.VMEM((_STEP, _EMBED), jnp.float32),
        pltpu.SemaphoreType.DMA,
        pltpu.SemaphoreType.DMA,
        pltpu.SemaphoreType.DMA,
        pltpu.SemaphoreType.DMA,
    ],
    compiler_params=pltpu.CompilerParams(use_tc_tiling_on_sc=True),
)
def _emb_kernel(x_hbm, table_hbm, pos_hbm, out_hbm,
                idx_v, idx_g, pos_v, buf0, buf1, st0, st1,
                gsem0, gsem1, ssem0, ssem1):
    wid = lax.axis_index("s") * _NC + lax.axis_index("c")
    pltpu.sync_copy(x_hbm.at[wid], idx_v)
    pltpu.sync_copy(pos_hbm, pos_v)

    @plsc.parallel_loop(0, _NSTEPS, step=1, unroll=2)
    def _(j):
        for k in range(_STEP // _L):
            sl = pl.ds(k * _L, _L)
            v = idx_v[j, sl]
            idx_g[j, sl] = jnp.left_shift(jnp.right_shift(v, 14), 13) + (v & 8191)

    bufs = (buf0, buf1)
    sts = (st0, st1)
    gsems = (gsem0, gsem1)
    ssems = (ssem0, ssem1)
    out_base = wid * _TOK_PER_W

    def issue_gather(g, buf, gsem):
        pltpu.async_copy(table_hbm.at[idx_g.at[g]], buf, gsem)

    def wait_gather(g, buf, gsem):
        pltpu.make_async_copy(table_hbm.at[idx_g.at[g]], buf, gsem).wait()

    def wait_store(st, ssem):
        pltpu.make_async_copy(st, out_hbm.at[pl.ds(0, _STEP)], ssem).wait()

    def add_and_store(g, cur, st, ssem):
        base_t = lax.rem(g * _STEP, _MAXLEN)

        @plsc.parallel_loop(0, _STEP, step=1, unroll=8)
        def _(r):
            t = base_t + r
            t = jnp.where(t >= _MAXLEN, t - _MAXLEN, t)
            vr = idx_v[g, pl.ds(r, 1)]
            half = (jnp.right_shift(vr[0], 13) & 1) * _EMBED
            for k in range(_EMBED // _L):
                st[r, pl.ds(k * _L, _L)] = (
                    cur[r, pl.ds(half + k * _L, _L)]
                    + pos_v[t, pl.ds(k * _L, _L)]
                )

        pltpu.async_copy(
            st, out_hbm.at[pl.ds(out_base + g * _STEP, _STEP)], ssem
        )

    issue_gather(0, buf0, gsem0)

    def outer(i, carry):
        for b in range(2):
            g = 2 * i + b
            cur, nxt = bufs[b], bufs[1 - b]
            st = sts[b]

            @pl.when(g + 1 < _NSTEPS)
            def _():
                issue_gather(g + 1, nxt, gsems[1 - b])

            @pl.when(g >= 2)
            def _():
                wait_store(st, ssems[b])

            wait_gather(g, cur, gsems[b])
            add_and_store(g, cur, st, ssems[b])
        return carry

    lax.fori_loop(0, _NSTEPS // 2, outer, 0)
    wait_store(st0, ssem0)
    wait_store(st1, ssem1)


def kernel(x, token_table, pos_table):
    x_r = x.reshape(_NW, _NSTEPS, _STEP)
    repacked = _repack_kernel(token_table.T)
    out = _emb_kernel(x_r, repacked, pos_table)
    return out.reshape(_BATCH, _MAXLEN, _EMBED)


# speedup vs baseline: 1.1473x; 1.0342x over previous
import functools

import jax
import jax.numpy as jnp
from jax import lax
from jax.experimental import pallas as pl
from jax.experimental.pallas import tpu as pltpu
from jax.experimental.pallas import tpu_sc as plsc

_MAXLEN = 200
_EMBED = 64
_BATCH = 1024
_VOCAB = 1000000
_NC, _NS, _L = 2, 16, 16
_NW = _NC * _NS
_TOK_PER_W = _BATCH * _MAXLEN // _NW
_STEP = 128
_NSTEPS = _TOK_PER_W // _STEP
_SPAN = 32768
_RROWS = 31 * 16384


def _repack_body(tt_ref, out_ref):
    out_ref[:, 0:_EMBED] = tt_ref[:, 0:_SPAN // 2].T
    out_ref[:, _EMBED:128] = tt_ref[:, _SPAN // 2:_SPAN].T


_repack_kernel = pl.pallas_call(
    _repack_body,
    out_shape=jax.ShapeDtypeStruct((_RROWS, 128), jnp.float32),
    grid=(pl.cdiv(_VOCAB, _SPAN),),
    in_specs=[pl.BlockSpec((_EMBED, _SPAN), lambda i: (0, i))],
    out_specs=pl.BlockSpec((_SPAN // 2, 128), lambda i: (i, 0)),
)


@functools.partial(
    pl.kernel,
    out_type=jax.ShapeDtypeStruct((_BATCH * _MAXLEN, _EMBED), jnp.float32),
    mesh=plsc.VectorSubcoreMesh(core_axis_name="c", subcore_axis_name="s"),
    scratch_types=[
        pltpu.VMEM((_NSTEPS, _STEP), jnp.int32),
        pltpu.VMEM((_NSTEPS, _STEP), jnp.int32),
        pltpu.VMEM((_MAXLEN, _EMBED), jnp.float32),
        pltpu.VMEM((_STEP, 128), jnp.float32),
        pltpu.VMEM((_STEP, 128), jnp.float32),
        pltpu.VMEM((_STEP, _EMBED), jnp.float32),
        pltpu.VMEM((_STEP, _EMBED), jnp.float32),
        pltpu.SemaphoreType.DMA,
        pltpu.SemaphoreType.DMA,
        pltpu.SemaphoreType.DMA,
        pltpu.SemaphoreType.DMA,
    ],
    compiler_params=pltpu.CompilerParams(use_tc_tiling_on_sc=True),
)
def _emb_kernel(x_hbm, table_hbm, pos_hbm, out_hbm,
                idx_v, idx_g, pos_v, buf0, buf1, st0, st1,
                gsem0, gsem1, ssem0, ssem1):
    wid = lax.axis_index("s") * _NC + lax.axis_index("c")
    pltpu.sync_copy(x_hbm.at[wid], idx_v)
    pltpu.sync_copy(pos_hbm, pos_v)

    @plsc.parallel_loop(0, _NSTEPS, step=1, unroll=2)
    def _(j):
        for k in range(_STEP // _L):
            sl = pl.ds(k * _L, _L)
            v = idx_v[j, sl]
            idx_g[j, sl] = jnp.left_shift(jnp.right_shift(v, 15), 14) + (v & 16383)

    bufs = (buf0, buf1)
    sts = (st0, st1)
    gsems = (gsem0, gsem1)
    ssems = (ssem0, ssem1)
    out_base = wid * _TOK_PER_W

    def issue_gather(g, buf, gsem):
        pltpu.async_copy(table_hbm.at[idx_g.at[g]], buf, gsem)

    def wait_gather(g, buf, gsem):
        pltpu.make_async_copy(table_hbm.at[idx_g.at[g]], buf, gsem).wait()

    def wait_store(st, ssem):
        pltpu.make_async_copy(st, out_hbm.at[pl.ds(0, _STEP)], ssem).wait()

    def add_and_store(g, cur, st, ssem):
        base_t = lax.rem(g * _STEP, _MAXLEN)

        @plsc.parallel_loop(0, _STEP, step=1, unroll=8)
        def _(r):
            t = base_t + r
            t = jnp.where(t >= _MAXLEN, t - _MAXLEN, t)
            vr = idx_v[g, pl.ds(r, 1)]
            half = (jnp.right_shift(vr[0], 14) & 1) * _EMBED
            for k in range(_EMBED // _L):
                st[r, pl.ds(k * _L, _L)] = (
                    cur[r, pl.ds(half + k * _L, _L)]
                    + pos_v[t, pl.ds(k * _L, _L)]
                )

        pltpu.async_copy(
            st, out_hbm.at[pl.ds(out_base + g * _STEP, _STEP)], ssem
        )

    issue_gather(0, buf0, gsem0)

    def outer(i, carry):
        for b in range(2):
            g = 2 * i + b
            cur, nxt = bufs[b], bufs[1 - b]
            st = sts[b]

            @pl.when(g + 1 < _NSTEPS)
            def _():
                issue_gather(g + 1, nxt, gsems[1 - b])

            @pl.when(g >= 2)
            def _():
                wait_store(st, ssems[b])

            wait_gather(g, cur, gsems[b])
            add_and_store(g, cur, st, ssems[b])
        return carry

    lax.fori_loop(0, _NSTEPS // 2, outer, 0)
    wait_store(st0, ssem0)
    wait_store(st1, ssem1)


def kernel(x, token_table, pos_table):
    x_r = x.reshape(_NW, _NSTEPS, _STEP)
    repacked = _repack_kernel(token_table.T)
    out = _emb_kernel(x_r, repacked, pos_table)
    return out.reshape(_BATCH, _MAXLEN, _EMBED)
